# baseline (speedup 1.0000x reference)
import jax
import jax.numpy as jnp
from jax.experimental import pallas as pl
from jax.experimental.pallas import tpu as pltpu

_B = 256
_K = 32
_IN = 128
_LAT = 256
_ND = 128
_ED = 128
_NA = 32
_CG = 32
_CR = _CG * _K
_GB = 64
_GRID = _B // _GB
_ROWS = _GB * _K


def _gn_chunk(theta,
              encW1_ref, encb1_ref, encW2_ref, encb2_ref,
              eW1a_ref, eW1b_ref, eb1_ref, eW2_ref, eb2_ref,
              nW1a_ref, nW1b_ref, nb1_ref, nW2_ref, nb2_ref,
              gW1a_ref, gW1b_ref, gb1_ref, gW2_ref, gb2_ref):
    f32 = jnp.float32

    def dot(x, w):
        return jnp.dot(x, w, preferred_element_type=f32)

    h = jnp.maximum(dot(theta, encW1_ref[...]) + encb1_ref[...], 0.0)
    h = dot(h, encW2_ref[...]) + encb2_ref[...]

    h = jnp.swapaxes(h.reshape(_CG, _K, _ND), 0, 1).reshape(_CR, _ND)

    A = dot(h, eW1a_ref[...]) + eb1_ref[...]
    S = dot(h, eW1b_ref[...])
    bf16 = jnp.bfloat16
    A3 = A.astype(bf16).reshape(_K, _CG, _LAT)
    S3 = S.astype(bf16).reshape(_K, _CG, _LAT)
    S_t = [S3[s] for s in range(_K)]
    zero_b = jnp.zeros((), bf16)

    R_rows = []
    for r in range(_K):
        ar = A3[r]
        accs = [None, None, None, None]
        for s in range(_K):
            t = jnp.maximum(ar + S_t[s], zero_b)
            i = s >> 3
            accs[i] = t if accs[i] is None else accs[i] + t
        self_t = jnp.maximum(ar + S_t[r], zero_b).astype(jnp.float32)
        acc = ((accs[0].astype(jnp.float32) + accs[1].astype(jnp.float32))
               + (accs[2].astype(jnp.float32) + accs[3].astype(jnp.float32))
               - self_t)
        R_rows.append(acc)
    R = jnp.stack(R_rows, axis=0)

    Rflat = R.reshape(_CR, _LAT)
    recv_mean = dot(Rflat, eW2_ref[...]) * (1.0 / (_K - 1)) + eb2_ref[...]

    z = jnp.maximum(dot(recv_mean, nW1a_ref[...]) + dot(h, nW1b_ref[...])
                    + nb1_ref[...], 0.0)
    v = dot(z, nW2_ref[...]) + nb2_ref[...]

    Rsum = jnp.sum(R, axis=0) * (1.0 / (_K * (_K - 1)))
    edge_agg = dot(Rsum, eW2_ref[...]) + eb2_ref[...]
    node_agg = jnp.mean(v.reshape(_K, _CG, _ND), axis=0)

    zg = jnp.maximum(dot(edge_agg, gW1a_ref[...]) + dot(node_agg, gW1b_ref[...])
                     + gb1_ref[...], 0.0)
    return dot(zg, gW2_ref[...]) + gb2_ref[...]


def _gn_block_kernel(theta_ref, *refs):
    out_ref = refs[-1]
    w_refs = refs[:-1]
    for c in range(_GB // _CG):
        g = _gn_chunk(theta_ref[c * _CR:(c + 1) * _CR, :], *w_refs)
        out_ref[c * _CG:(c + 1) * _CG, :] = g


def _full(shape):
    return pl.BlockSpec(shape, lambda i: (0,) * len(shape))


@jax.jit
def kernel(theta, enc_W1, enc_b1, enc_W2, enc_b2,
           edge_W1, edge_b1, edge_W2, edge_b2,
           node_W1, node_b1, node_W2, node_b2,
           glob_W1, glob_b1, glob_W2, glob_b2):
    eW1a, eW1b = edge_W1[:_ND], edge_W1[_ND:]
    nW1a, nW1b = node_W1[:_ED], node_W1[_ED:]
    gW1a, gW1b = glob_W1[:_ED], glob_W1[_ED:]
    b = lambda x: x.reshape(1, -1)

    return pl.pallas_call(
        _gn_block_kernel,
        grid=(_GRID,),
        in_specs=[
            pl.BlockSpec((_ROWS, _IN), lambda i: (i, 0)),
            _full((_IN, _LAT)), _full((1, _LAT)),
            _full((_LAT, _ND)), _full((1, _ND)),
            _full((_ND, _LAT)), _full((_ND, _LAT)), _full((1, _LAT)),
            _full((_LAT, _ED)), _full((1, _ED)),
            _full((_ED, _LAT)), _full((_ND, _LAT)), _full((1, _LAT)),
            _full((_LAT, _ND)), _full((1, _ND)),
            _full((_ED, _LAT)), _full((_ND, _LAT)), _full((1, _LAT)),
            _full((_LAT, _NA)), _full((1, _NA)),
        ],
        out_specs=pl.BlockSpec((_GB, _NA), lambda i: (i, 0)),
        out_shape=jax.ShapeDtypeStruct((_B, _NA), jnp.float32),
        compiler_params=pltpu.CompilerParams(
            dimension_semantics=("arbitrary",),
        ),
    )(theta,
      enc_W1, b(enc_b1), enc_W2, b(enc_b2),
      eW1a, eW1b, b(edge_b1), edge_W2, b(edge_b2),
      nW1a, nW1b, b(node_b1), node_W2, b(node_b2),
      gW1a, gW1b, b(glob_b1), glob_W2, b(glob_b2))


# speedup vs baseline: 1.0812x; 1.0812x over previous
import jax
import jax.numpy as jnp
from jax.experimental import pallas as pl
from jax.experimental.pallas import tpu as pltpu

_B = 256
_K = 32
_IN = 128
_LAT = 256
_ND = 128
_ED = 128
_NA = 32
_CG = 32
_CR = _CG * _K
_GB = 64
_GRID = _B // _GB
_ROWS = _GB * _K


def _dot(x, w):
    return jnp.dot(x, w, preferred_element_type=jnp.float32)


def _gn_front(theta, encW1_ref, encb1_ref, encW2_ref, encb2_ref,
              eW1a_ref, eW1b_ref, eb1_ref):
    bf16 = jnp.bfloat16
    h = jnp.maximum(_dot(theta, encW1_ref[...]) + encb1_ref[...], 0.0)
    h = _dot(h, encW2_ref[...]) + encb2_ref[...]

    h = jnp.swapaxes(h.reshape(_CG, _K, _ND), 0, 1).reshape(_CR, _ND)

    hb = h.astype(bf16)
    A = _dot(hb, eW1a_ref[...].astype(bf16)) + eb1_ref[...]
    S = _dot(hb, eW1b_ref[...].astype(bf16))
    A3 = A.astype(bf16).reshape(_K, _CG, _LAT)
    S3 = S.astype(bf16).reshape(_K, _CG, _LAT)
    return h, A3, S3


def _gn_back(h, A3, S3,
             eW2_ref, eb2_ref,
             nW1a_ref, nW1b_ref, nb1_ref, nW2_ref, nb2_ref,
             gW1a_ref, gW1b_ref, gb1_ref, gW2_ref, gb2_ref):
    f32 = jnp.float32
    dot = _dot
    bf16 = jnp.bfloat16
    S_t = [S3[s] for s in range(_K)]
    zero_b = jnp.zeros((), bf16)

    R_rows = []
    for r in range(_K):
        ar = A3[r]
        accs = [None, None, None, None]
        for s in range(_K):
            t = jnp.maximum(ar + S_t[s], zero_b)
            i = s >> 3
            accs[i] = t if accs[i] is None else accs[i] + t
        self_t = jnp.maximum(ar + S_t[r], zero_b).astype(jnp.float32)
        acc = ((accs[0].astype(jnp.float32) + accs[1].astype(jnp.float32))
               + (accs[2].astype(jnp.float32) + accs[3].astype(jnp.float32))
               - self_t)
        R_rows.append(acc)
    R = jnp.stack(R_rows, axis=0)

    Rflat = R.reshape(_CR, _LAT)
    recv_mean = dot(Rflat, eW2_ref[...]) * (1.0 / (_K - 1)) + eb2_ref[...]

    z = jnp.maximum(dot(recv_mean, nW1a_ref[...]) + dot(h, nW1b_ref[...])
                    + nb1_ref[...], 0.0)
    v = dot(z, nW2_ref[...]) + nb2_ref[...]

    Rsum = jnp.sum(R, axis=0) * (1.0 / (_K * (_K - 1)))
    edge_agg = dot(Rsum, eW2_ref[...]) + eb2_ref[...]
    node_agg = jnp.mean(v.reshape(_K, _CG, _ND), axis=0)

    zg = jnp.maximum(dot(edge_agg, gW1a_ref[...]) + dot(node_agg, gW1b_ref[...])
                     + gb1_ref[...], 0.0)
    return dot(zg, gW2_ref[...]) + gb2_ref[...]


def _gn_block_kernel(theta_ref,
                     encW1_ref, encb1_ref, encW2_ref, encb2_ref,
                     eW1a_ref, eW1b_ref, eb1_ref, eW2_ref, eb2_ref,
                     nW1a_ref, nW1b_ref, nb1_ref, nW2_ref, nb2_ref,
                     gW1a_ref, gW1b_ref, gb1_ref, gW2_ref, gb2_ref,
                     out_ref):
    fronts = []
    for c in range(_GB // _CG):
        fronts.append(_gn_front(theta_ref[c * _CR:(c + 1) * _CR, :],
                                encW1_ref, encb1_ref, encW2_ref, encb2_ref,
                                eW1a_ref, eW1b_ref, eb1_ref))
    for c in range(_GB // _CG):
        h, A3, S3 = fronts[c]
        g = _gn_back(h, A3, S3,
                     eW2_ref, eb2_ref,
                     nW1a_ref, nW1b_ref, nb1_ref, nW2_ref, nb2_ref,
                     gW1a_ref, gW1b_ref, gb1_ref, gW2_ref, gb2_ref)
        out_ref[c * _CG:(c + 1) * _CG, :] = g


def _full(shape):
    return pl.BlockSpec(shape, lambda i: (0,) * len(shape))


@jax.jit
def kernel(theta, enc_W1, enc_b1, enc_W2, enc_b2,
           edge_W1, edge_b1, edge_W2, edge_b2,
           node_W1, node_b1, node_W2, node_b2,
           glob_W1, glob_b1, glob_W2, glob_b2):
    eW1a, eW1b = edge_W1[:_ND], edge_W1[_ND:]
    nW1a, nW1b = node_W1[:_ED], node_W1[_ED:]
    gW1a, gW1b = glob_W1[:_ED], glob_W1[_ED:]
    b = lambda x: x.reshape(1, -1)

    return pl.pallas_call(
        _gn_block_kernel,
        grid=(_GRID,),
        in_specs=[
            pl.BlockSpec((_ROWS, _IN), lambda i: (i, 0)),
            _full((_IN, _LAT)), _full((1, _LAT)),
            _full((_LAT, _ND)), _full((1, _ND)),
            _full((_ND, _LAT)), _full((_ND, _LAT)), _full((1, _LAT)),
            _full((_LAT, _ED)), _full((1, _ED)),
            _full((_ED, _LAT)), _full((_ND, _LAT)), _full((1, _LAT)),
            _full((_LAT, _ND)), _full((1, _ND)),
            _full((_ED, _LAT)), _full((_ND, _LAT)), _full((1, _LAT)),
            _full((_LAT, _NA)), _full((1, _NA)),
        ],
        out_specs=pl.BlockSpec((_GB, _NA), lambda i: (i, 0)),
        out_shape=jax.ShapeDtypeStruct((_B, _NA), jnp.float32),
        compiler_params=pltpu.CompilerParams(
            dimension_semantics=("arbitrary",),
        ),
    )(theta,
      enc_W1, b(enc_b1), enc_W2, b(enc_b2),
      eW1a, eW1b, b(edge_b1), edge_W2, b(edge_b2),
      nW1a, nW1b, b(node_b1), node_W2, b(node_b2),
      gW1a, gW1b, b(glob_b1), glob_W2, b(glob_b2))


# speedup vs baseline: 1.1320x; 1.0470x over previous
import jax
import jax.numpy as jnp
from jax.experimental import pallas as pl
from jax.experimental.pallas import tpu as pltpu

_B = 256
_K = 32
_IN = 128
_LAT = 256
_ND = 128
_ED = 128
_NA = 32
_CG = 32
_CR = _CG * _K
_GB = 256
_GRID = _B // _GB
_ROWS = _GB * _K


def _dot(x, w):
    return jnp.dot(x, w, preferred_element_type=jnp.float32)


def _gn_front(theta, encW1_ref, encb1_ref, encW2_ref, encb2_ref,
              eW1a_ref, eW1b_ref, eb1_ref):
    bf16 = jnp.bfloat16
    h = jnp.maximum(_dot(theta, encW1_ref[...]) + encb1_ref[...], 0.0)
    h = _dot(h, encW2_ref[...]) + encb2_ref[...]

    h = jnp.swapaxes(h.reshape(_CG, _K, _ND), 0, 1).reshape(_CR, _ND)

    hb = h.astype(bf16)
    A = _dot(hb, eW1a_ref[...].astype(bf16)) + eb1_ref[...]
    S = _dot(hb, eW1b_ref[...].astype(bf16))
    A3 = A.astype(bf16).reshape(_K, _CG, _LAT)
    S3 = S.astype(bf16).reshape(_K, _CG, _LAT)
    return h, A3, S3


def _gn_back(h, A3, S3,
             eW2_ref, eb2_ref,
             nW1a_ref, nW1b_ref, nb1_ref, nW2_ref, nb2_ref,
             gW1a_ref, gW1b_ref, gb1_ref, gW2_ref, gb2_ref):
    f32 = jnp.float32
    dot = _dot
    bf16 = jnp.bfloat16
    S_t = [S3[s] for s in range(_K)]
    zero_b = jnp.zeros((), bf16)

    R_rows = []
    for r in range(_K):
        ar = A3[r]
        accs = [None, None]
        for s in range(_K):
            t = jnp.maximum(ar + S_t[s], zero_b)
            i = s >> 4
            accs[i] = t if accs[i] is None else accs[i] + t
        self_t = jnp.maximum(ar + S_t[r], zero_b).astype(jnp.float32)
        acc = (accs[0].astype(jnp.float32) + accs[1].astype(jnp.float32)
               - self_t)
        R_rows.append(acc)
    R = jnp.stack(R_rows, axis=0)

    Rflat = R.reshape(_CR, _LAT)
    recv_mean = dot(Rflat, eW2_ref[...]) * (1.0 / (_K - 1)) + eb2_ref[...]

    z = jnp.maximum(dot(recv_mean, nW1a_ref[...]) + dot(h, nW1b_ref[...])
                    + nb1_ref[...], 0.0)
    v = dot(z, nW2_ref[...]) + nb2_ref[...]

    Rsum = jnp.sum(R, axis=0) * (1.0 / (_K * (_K - 1)))
    edge_agg = dot(Rsum, eW2_ref[...]) + eb2_ref[...]
    node_agg = jnp.mean(v.reshape(_K, _CG, _ND), axis=0)

    zg = jnp.maximum(dot(edge_agg, gW1a_ref[...]) + dot(node_agg, gW1b_ref[...])
                     + gb1_ref[...], 0.0)
    return dot(zg, gW2_ref[...]) + gb2_ref[...]


def _gn_block_kernel(theta_ref,
                     encW1_ref, encb1_ref, encW2_ref, encb2_ref,
                     eW1a_ref, eW1b_ref, eb1_ref, eW2_ref, eb2_ref,
                     nW1a_ref, nW1b_ref, nb1_ref, nW2_ref, nb2_ref,
                     gW1a_ref, gW1b_ref, gb1_ref, gW2_ref, gb2_ref,
                     out_ref):
    fronts = []
    for c in range(_GB // _CG):
        fronts.append(_gn_front(theta_ref[c * _CR:(c + 1) * _CR, :],
                                encW1_ref, encb1_ref, encW2_ref, encb2_ref,
                                eW1a_ref, eW1b_ref, eb1_ref))
    for c in range(_GB // _CG):
        h, A3, S3 = fronts[c]
        g = _gn_back(h, A3, S3,
                     eW2_ref, eb2_ref,
                     nW1a_ref, nW1b_ref, nb1_ref, nW2_ref, nb2_ref,
                     gW1a_ref, gW1b_ref, gb1_ref, gW2_ref, gb2_ref)
        out_ref[c * _CG:(c + 1) * _CG, :] = g


def _full(shape):
    return pl.BlockSpec(shape, lambda i: (0,) * len(shape))


@jax.jit
def kernel(theta, enc_W1, enc_b1, enc_W2, enc_b2,
           edge_W1, edge_b1, edge_W2, edge_b2,
           node_W1, node_b1, node_W2, node_b2,
           glob_W1, glob_b1, glob_W2, glob_b2):
    eW1a, eW1b = edge_W1[:_ND], edge_W1[_ND:]
    nW1a, nW1b = node_W1[:_ED], node_W1[_ED:]
    gW1a, gW1b = glob_W1[:_ED], glob_W1[_ED:]
    b = lambda x: x.reshape(1, -1)

    return pl.pallas_call(
        _gn_block_kernel,
        grid=(_GRID,),
        in_specs=[
            pl.BlockSpec((_ROWS, _IN), lambda i: (i, 0)),
            _full((_IN, _LAT)), _full((1, _LAT)),
            _full((_LAT, _ND)), _full((1, _ND)),
            _full((_ND, _LAT)), _full((_ND, _LAT)), _full((1, _LAT)),
            _full((_LAT, _ED)), _full((1, _ED)),
            _full((_ED, _LAT)), _full((_ND, _LAT)), _full((1, _LAT)),
            _full((_LAT, _ND)), _full((1, _ND)),
            _full((_ED, _LAT)), _full((_ND, _LAT)), _full((1, _LAT)),
            _full((_LAT, _NA)), _full((1, _NA)),
        ],
        out_specs=pl.BlockSpec((_GB, _NA), lambda i: (i, 0)),
        out_shape=jax.ShapeDtypeStruct((_B, _NA), jnp.float32),
        compiler_params=pltpu.CompilerParams(
            dimension_semantics=("arbitrary",),
        ),
    )(theta,
      enc_W1, b(enc_b1), enc_W2, b(enc_b2),
      eW1a, eW1b, b(edge_b1), edge_W2, b(edge_b2),
      nW1a, nW1b, b(node_b1), node_W2, b(node_b2),
      gW1a, gW1b, b(glob_b1), glob_W2, b(glob_b2))


# speedup vs baseline: 1.1340x; 1.0018x over previous
import jax
import jax.numpy as jnp
from jax.experimental import pallas as pl
from jax.experimental.pallas import tpu as pltpu

_B = 256
_K = 32
_IN = 128
_LAT = 256
_ND = 128
_ED = 128
_NA = 32
_CG = 32
_CR = _CG * _K
_GB = 128
_GRID = _B // _GB
_ROWS = _GB * _K


def _dot(x, w):
    return jnp.dot(x, w, preferred_element_type=jnp.float32)


def _gn_front(theta, encW1_ref, encb1_ref, encW2_ref, encb2_ref,
              eW1a_ref, eW1b_ref, eb1_ref):
    bf16 = jnp.bfloat16
    h = jnp.maximum(_dot(theta, encW1_ref[...]) + encb1_ref[...], 0.0)
    h = _dot(h, encW2_ref[...]) + encb2_ref[...]

    h = jnp.swapaxes(h.reshape(_CG, _K, _ND), 0, 1).reshape(_CR, _ND)

    hb = h.astype(bf16)
    A = _dot(hb, eW1a_ref[...].astype(bf16)) + eb1_ref[...]
    S = _dot(hb, eW1b_ref[...].astype(bf16))
    A3 = A.astype(bf16).reshape(_K, _CG, _LAT)
    S3 = S.astype(bf16).reshape(_K, _CG, _LAT)
    return h, A3, S3


def _gn_back(h, A3, S3,
             eW2_ref, eb2_ref,
             nW1a_ref, nW1b_ref, nb1_ref, nW2_ref, nb2_ref,
             gW1a_ref, gW1b_ref, gb1_ref, gW2_ref, gb2_ref):
    f32 = jnp.float32
    dot = _dot
    bf16 = jnp.bfloat16
    S_t = [S3[s] for s in range(_K)]
    zero_b = jnp.zeros((), bf16)

    R_rows = []
    for r in range(_K):
        ar = A3[r]
        accs = [None, None]
        for s in range(_K):
            t = jnp.maximum(ar + S_t[s], zero_b)
            i = s >> 4
            accs[i] = t if accs[i] is None else accs[i] + t
        self_t = jnp.maximum(ar + S_t[r], zero_b).astype(jnp.float32)
        acc = (accs[0].astype(jnp.float32) + accs[1].astype(jnp.float32)
               - self_t)
        R_rows.append(acc)
    R = jnp.stack(R_rows, axis=0)

    Rflat = R.reshape(_CR, _LAT)
    recv_mean = dot(Rflat, eW2_ref[...]) * (1.0 / (_K - 1)) + eb2_ref[...]

    z = jnp.maximum(dot(recv_mean, nW1a_ref[...]) + dot(h, nW1b_ref[...])
                    + nb1_ref[...], 0.0)
    v = dot(z, nW2_ref[...]) + nb2_ref[...]

    Rsum = jnp.sum(R, axis=0) * (1.0 / (_K * (_K - 1)))
    edge_agg = dot(Rsum, eW2_ref[...]) + eb2_ref[...]
    node_agg = jnp.mean(v.reshape(_K, _CG, _ND), axis=0)

    zg = jnp.maximum(dot(edge_agg, gW1a_ref[...]) + dot(node_agg, gW1b_ref[...])
                     + gb1_ref[...], 0.0)
    return dot(zg, gW2_ref[...]) + gb2_ref[...]


def _gn_block_kernel(theta_ref,
                     encW1_ref, encb1_ref, encW2_ref, encb2_ref,
                     eW1a_ref, eW1b_ref, eb1_ref, eW2_ref, eb2_ref,
                     nW1a_ref, nW1b_ref, nb1_ref, nW2_ref, nb2_ref,
                     gW1a_ref, gW1b_ref, gb1_ref, gW2_ref, gb2_ref,
                     out_ref):
    fronts = []
    for c in range(_GB // _CG):
        fronts.append(_gn_front(theta_ref[c * _CR:(c + 1) * _CR, :],
                                encW1_ref, encb1_ref, encW2_ref, encb2_ref,
                                eW1a_ref, eW1b_ref, eb1_ref))
    for c in range(_GB // _CG):
        h, A3, S3 = fronts[c]
        g = _gn_back(h, A3, S3,
                     eW2_ref, eb2_ref,
                     nW1a_ref, nW1b_ref, nb1_ref, nW2_ref, nb2_ref,
                     gW1a_ref, gW1b_ref, gb1_ref, gW2_ref, gb2_ref)
        out_ref[c * _CG:(c + 1) * _CG, :] = g


def _full(shape):
    return pl.BlockSpec(shape, lambda i: (0,) * len(shape))


@jax.jit
def kernel(theta, enc_W1, enc_b1, enc_W2, enc_b2,
           edge_W1, edge_b1, edge_W2, edge_b2,
           node_W1, node_b1, node_W2, node_b2,
           glob_W1, glob_b1, glob_W2, glob_b2):
    eW1a, eW1b = edge_W1[:_ND], edge_W1[_ND:]
    nW1a, nW1b = node_W1[:_ED], node_W1[_ED:]
    gW1a, gW1b = glob_W1[:_ED], glob_W1[_ED:]
    b = lambda x: x.reshape(1, -1)

    return pl.pallas_call(
        _gn_block_kernel,
        grid=(_GRID,),
        in_specs=[
            pl.BlockSpec((_ROWS, _IN), lambda i: (i, 0)),
            _full((_IN, _LAT)), _full((1, _LAT)),
            _full((_LAT, _ND)), _full((1, _ND)),
            _full((_ND, _LAT)), _full((_ND, _LAT)), _full((1, _LAT)),
            _full((_LAT, _ED)), _full((1, _ED)),
            _full((_ED, _LAT)), _full((_ND, _LAT)), _full((1, _LAT)),
            _full((_LAT, _ND)), _full((1, _ND)),
            _full((_ED, _LAT)), _full((_ND, _LAT)), _full((1, _LAT)),
            _full((_LAT, _NA)), _full((1, _NA)),
        ],
        out_specs=pl.BlockSpec((_GB, _NA), lambda i: (i, 0)),
        out_shape=jax.ShapeDtypeStruct((_B, _NA), jnp.float32),
        compiler_params=pltpu.CompilerParams(
            dimension_semantics=("arbitrary",),
        ),
    )(theta,
      enc_W1, b(enc_b1), enc_W2, b(enc_b2),
      eW1a, eW1b, b(edge_b1), edge_W2, b(edge_b2),
      nW1a, nW1b, b(node_b1), node_W2, b(node_b2),
      gW1a, gW1b, b(glob_b1), glob_W2, b(glob_b2))


# speedup vs baseline: 1.1579x; 1.0210x over previous
import jax
import jax.numpy as jnp
from jax.experimental import pallas as pl
from jax.experimental.pallas import tpu as pltpu

_B = 256
_K = 32
_IN = 128
_LAT = 256
_ND = 128
_ED = 128
_NA = 32
_CG = 32
_CR = _CG * _K
_GB = 128
_GRID = _B // _GB
_ROWS = _GB * _K


def _dot(x, w):
    return jnp.dot(x, w, preferred_element_type=jnp.float32)


def _gn_front(theta, encW1_ref, encb1_ref, encW2_ref, encb2_ref,
              eW1a_ref, eW1b_ref, eb1_ref):
    bf16 = jnp.bfloat16
    h = jnp.maximum(_dot(theta, encW1_ref[...]) + encb1_ref[...], 0.0)
    h = _dot(h, encW2_ref[...]) + encb2_ref[...]

    h = jnp.swapaxes(h.reshape(_CG, _K, _ND), 0, 1).reshape(_CR, _ND)

    hb = h.astype(bf16)
    A = _dot(hb, eW1a_ref[...].astype(bf16)) + eb1_ref[...]
    S = _dot(hb, eW1b_ref[...].astype(bf16))
    A3 = A.astype(bf16).reshape(_K, _CG, _LAT)
    S3 = S.astype(bf16).reshape(_K, _CG, _LAT)
    return h, A3, S3


def _gn_back(h, A3, S3,
             eW2_ref, eb2_ref,
             nW1a_ref, nW1b_ref, nb1_ref, nW2_ref, nb2_ref,
             gW1a_ref, gW1b_ref, gb1_ref, gW2_ref, gb2_ref):
    f32 = jnp.float32
    dot = _dot
    bf16 = jnp.bfloat16
    S_t = [S3[s] for s in range(_K)]
    zero_b = jnp.zeros((), bf16)

    R_rows = []
    for r in range(_K):
        ar = A3[r]
        accs = [None, None]
        for s in range(_K):
            t = jnp.maximum(ar + S_t[s], zero_b)
            i = s >> 4
            accs[i] = t if accs[i] is None else accs[i] + t
        self_t = jnp.maximum(ar + S_t[r], zero_b)
        R_rows.append((accs[0] + accs[1]) - self_t)
    R = jnp.stack(R_rows, axis=0)

    Rflat = R.reshape(_CR, _LAT)
    recv_mean = (dot(Rflat, eW2_ref[...].astype(bf16)) * (1.0 / (_K - 1))
                 + eb2_ref[...])

    z = jnp.maximum(dot(recv_mean, nW1a_ref[...]) + dot(h, nW1b_ref[...])
                    + nb1_ref[...], 0.0)
    v = dot(z, nW2_ref[...]) + nb2_ref[...]

    Rsum = (jnp.sum(R.astype(f32), axis=0)
            * (1.0 / (_K * (_K - 1))))
    edge_agg = dot(Rsum, eW2_ref[...]) + eb2_ref[...]
    node_agg = jnp.mean(v.reshape(_K, _CG, _ND), axis=0)

    zg = jnp.maximum(dot(edge_agg, gW1a_ref[...]) + dot(node_agg, gW1b_ref[...])
                     + gb1_ref[...], 0.0)
    return dot(zg, gW2_ref[...]) + gb2_ref[...]


def _gn_block_kernel(theta_ref,
                     encW1_ref, encb1_ref, encW2_ref, encb2_ref,
                     eW1a_ref, eW1b_ref, eb1_ref, eW2_ref, eb2_ref,
                     nW1a_ref, nW1b_ref, nb1_ref, nW2_ref, nb2_ref,
                     gW1a_ref, gW1b_ref, gb1_ref, gW2_ref, gb2_ref,
                     out_ref):
    fronts = []
    for c in range(_GB // _CG):
        fronts.append(_gn_front(theta_ref[c * _CR:(c + 1) * _CR, :],
                                encW1_ref, encb1_ref, encW2_ref, encb2_ref,
                                eW1a_ref, eW1b_ref, eb1_ref))
    for c in range(_GB // _CG):
        h, A3, S3 = fronts[c]
        g = _gn_back(h, A3, S3,
                     eW2_ref, eb2_ref,
                     nW1a_ref, nW1b_ref, nb1_ref, nW2_ref, nb2_ref,
                     gW1a_ref, gW1b_ref, gb1_ref, gW2_ref, gb2_ref)
        out_ref[c * _CG:(c + 1) * _CG, :] = g


def _full(shape):
    return pl.BlockSpec(shape, lambda i: (0,) * len(shape))


@jax.jit
def kernel(theta, enc_W1, enc_b1, enc_W2, enc_b2,
           edge_W1, edge_b1, edge_W2, edge_b2,
           node_W1, node_b1, node_W2, node_b2,
           glob_W1, glob_b1, glob_W2, glob_b2):
    eW1a, eW1b = edge_W1[:_ND], edge_W1[_ND:]
    nW1a, nW1b = node_W1[:_ED], node_W1[_ED:]
    gW1a, gW1b = glob_W1[:_ED], glob_W1[_ED:]
    b = lambda x: x.reshape(1, -1)

    return pl.pallas_call(
        _gn_block_kernel,
        grid=(_GRID,),
        in_specs=[
            pl.BlockSpec((_ROWS, _IN), lambda i: (i, 0)),
            _full((_IN, _LAT)), _full((1, _LAT)),
            _full((_LAT, _ND)), _full((1, _ND)),
            _full((_ND, _LAT)), _full((_ND, _LAT)), _full((1, _LAT)),
            _full((_LAT, _ED)), _full((1, _ED)),
            _full((_ED, _LAT)), _full((_ND, _LAT)), _full((1, _LAT)),
            _full((_LAT, _ND)), _full((1, _ND)),
            _full((_ED, _LAT)), _full((_ND, _LAT)), _full((1, _LAT)),
            _full((_LAT, _NA)), _full((1, _NA)),
        ],
        out_specs=pl.BlockSpec((_GB, _NA), lambda i: (i, 0)),
        out_shape=jax.ShapeDtypeStruct((_B, _NA), jnp.float32),
        compiler_params=pltpu.CompilerParams(
            dimension_semantics=("arbitrary",),
        ),
    )(theta,
      enc_W1, b(enc_b1), enc_W2, b(enc_b2),
      eW1a, eW1b, b(edge_b1), edge_W2, b(edge_b2),
      nW1a, nW1b, b(node_b1), node_W2, b(node_b2),
      gW1a, gW1b, b(glob_b1), glob_W2, b(glob_b2))


# speedup vs baseline: 1.1585x; 1.0005x over previous
import jax
import jax.numpy as jnp
from jax.experimental import pallas as pl
from jax.experimental.pallas import tpu as pltpu

_B = 256
_K = 32
_IN = 128
_LAT = 256
_ND = 128
_ED = 128
_NA = 32
_CG = 32
_CR = _CG * _K
_GB = 128
_GRID = _B // _GB
_ROWS = _GB * _K


def _dot(x, w):
    return jnp.dot(x, w, preferred_element_type=jnp.float32)


def _gn_front(theta, encW1_ref, encb1_ref, encW2_ref, encb2_ref,
              eW1a_ref, eW1b_ref, eb1_ref):
    bf16 = jnp.bfloat16
    h = jnp.maximum(_dot(theta, encW1_ref[...]) + encb1_ref[...], 0.0)
    h = _dot(h, encW2_ref[...]) + encb2_ref[...]

    h = jnp.swapaxes(h.reshape(_CG, _K, _ND), 0, 1).reshape(_CR, _ND)

    hb = h.astype(bf16)
    A = _dot(hb, eW1a_ref[...].astype(bf16)) + eb1_ref[...]
    S = _dot(hb, eW1b_ref[...].astype(bf16))
    A3 = A.astype(bf16).reshape(_K, _CG, _LAT)
    S3 = S.astype(bf16).reshape(_K, _CG, _LAT)
    return h, A3, S3


def _gn_back(h, A3, S3,
             eW2_ref, eb2_ref,
             nW1a_ref, nW1b_ref, nb1_ref, nW2_ref, nb2_ref,
             gW1a_ref, gW1b_ref, gb1_ref, gW2_ref, gb2_ref):
    f32 = jnp.float32
    dot = _dot
    bf16 = jnp.bfloat16
    S_t = [S3[s] for s in range(_K)]
    zero_b = jnp.zeros((), bf16)

    R_rows = []
    for r in range(_K):
        ar = A3[r]
        accs = [None, None]
        for s in range(_K):
            t = jnp.maximum(ar + S_t[s], zero_b)
            i = s >> 4
            accs[i] = t if accs[i] is None else accs[i] + t
        self_t = jnp.maximum(ar + S_t[r], zero_b)
        R_rows.append((accs[0] + accs[1]) - self_t)
    R = jnp.stack(R_rows, axis=0)

    Rflat = R.reshape(_CR, _LAT)
    recv_mean = (dot(Rflat, eW2_ref[...].astype(bf16)) * (1.0 / (_K - 1))
                 + eb2_ref[...])

    z = jnp.maximum(dot(recv_mean, nW1a_ref[...]) + dot(h, nW1b_ref[...])
                    + nb1_ref[...], 0.0)
    v = dot(z, nW2_ref[...]) + nb2_ref[...]

    Rsum = (jnp.sum(R.astype(f32), axis=0)
            * (1.0 / (_K * (_K - 1))))
    edge_agg = dot(Rsum, eW2_ref[...]) + eb2_ref[...]
    node_agg = jnp.mean(v.reshape(_K, _CG, _ND), axis=0)

    zg = jnp.maximum(dot(edge_agg, gW1a_ref[...]) + dot(node_agg, gW1b_ref[...])
                     + gb1_ref[...], 0.0)
    return dot(zg, gW2_ref[...]) + gb2_ref[...]


def _gn_block_kernel(theta_ref,
                     encW1_ref, encb1_ref, encW2_ref, encb2_ref,
                     eW1a_ref, eW1b_ref, eb1_ref, eW2_ref, eb2_ref,
                     nW1a_ref, nW1b_ref, nb1_ref, nW2_ref, nb2_ref,
                     gW1a_ref, gW1b_ref, gb1_ref, gW2_ref, gb2_ref,
                     out_ref):
    fronts = []
    for c in range(_GB // _CG):
        fronts.append(_gn_front(theta_ref[c * _CR:(c + 1) * _CR, :],
                                encW1_ref, encb1_ref, encW2_ref, encb2_ref,
                                eW1a_ref, eW1b_ref, eb1_ref))
    for c in range(_GB // _CG):
        h, A3, S3 = fronts[c]
        g = _gn_back(h, A3, S3,
                     eW2_ref, eb2_ref,
                     nW1a_ref, nW1b_ref, nb1_ref, nW2_ref, nb2_ref,
                     gW1a_ref, gW1b_ref, gb1_ref, gW2_ref, gb2_ref)
        out_ref[c * _CG:(c + 1) * _CG, :] = g


def _full(shape):
    return pl.BlockSpec(shape, lambda i: (0,) * len(shape))


@jax.jit
def kernel(theta, enc_W1, enc_b1, enc_W2, enc_b2,
           edge_W1, edge_b1, edge_W2, edge_b2,
           node_W1, node_b1, node_W2, node_b2,
           glob_W1, glob_b1, glob_W2, glob_b2):
    eW1a, eW1b = edge_W1[:_ND], edge_W1[_ND:]
    nW1a, nW1b = node_W1[:_ED], node_W1[_ED:]
    gW1a, gW1b = glob_W1[:_ED], glob_W1[_ED:]
    b = lambda x: x.reshape(1, -1)

    return pl.pallas_call(
        _gn_block_kernel,
        grid=(_GRID,),
        in_specs=[
            pl.BlockSpec((_ROWS, _IN), lambda i: (i, 0)),
            _full((_IN, _LAT)), _full((1, _LAT)),
            _full((_LAT, _ND)), _full((1, _ND)),
            _full((_ND, _LAT)), _full((_ND, _LAT)), _full((1, _LAT)),
            _full((_LAT, _ED)), _full((1, _ED)),
            _full((_ED, _LAT)), _full((_ND, _LAT)), _full((1, _LAT)),
            _full((_LAT, _ND)), _full((1, _ND)),
            _full((_ED, _LAT)), _full((_ND, _LAT)), _full((1, _LAT)),
            _full((_LAT, _NA)), _full((1, _NA)),
        ],
        out_specs=pl.BlockSpec((_GB, _NA), lambda i: (i, 0)),
        out_shape=jax.ShapeDtypeStruct((_B, _NA), jnp.float32),
        compiler_params=pltpu.CompilerParams(
            dimension_semantics=("parallel",),
        ),
    )(theta,
      enc_W1, b(enc_b1), enc_W2, b(enc_b2),
      eW1a, eW1b, b(edge_b1), edge_W2, b(edge_b2),
      nW1a, nW1b, b(node_b1), node_W2, b(node_b2),
      gW1a, gW1b, b(glob_b1), glob_W2, b(glob_b2))
